# baseline (speedup 1.0000x reference)
import functools

import jax
import jax.numpy as jnp
from jax import lax
from jax.experimental import pallas as pl
from jax.experimental.pallas import tpu as pltpu
from jax.experimental.pallas import tpu_sc as plsc

N = 10000
E = 320000
D = 128
V = 1000
G = 256

NW = 32
R = 313
NPAD = NW * R
CHK = 2000
BB = 512
ACCN = BB + CHK + 16

KC = 256
NPAD_C = N + KC


def _agg_body(wid_h, src_h, dst_h, tab_h, rst_h,
              wid_v, agg_v, msgs_v, dstc_v, srcc_v, dstc2_v, srcc2_v,
              accd_v, accw_v, widb_v, tmp_v, sem, sem2, sem3):
    w = lax.axis_index("s") * 2 + lax.axis_index("c")
    lo = w * R
    hi = jnp.minimum(lo + R, N)
    neg16 = jnp.full((16,), -jnp.inf, jnp.float32)
    zero16 = jnp.zeros((16,), jnp.int32)

    pltpu.sync_copy(wid_h, wid_v.at[pl.ds(0, N)])
    wid_v[pl.ds(N, 16)] = zero16
    wid_v[pl.ds(N + 16, 16)] = zero16

    def _init_accw(i, _):
        accw_v[pl.ds(i * 16, 16)] = zero16
        return 0
    lax.fori_loop(0, ACCN // 16, _init_accw, 0)

    def _init_agg(i, _):
        agg_v[pl.ds(i * 16, 16)] = neg16
        return 0
    lax.fori_loop(0, (R * D) // 16, _init_agg, 0)

    lanes = lax.iota(jnp.int32, 16)

    def _process_batch(n):
        def _cp(i, _):
            widb_v[pl.ds(i * 16, 16)] = accw_v[pl.ds(i * 16, 16)]
            return 0
        lax.fori_loop(0, BB // 16, _cp, 0)
        pltpu.async_copy(tab_h.at[widb_v], msgs_v, sem).wait()

        def _rmw1(e, d):
            base = d * D
            for k in range(8):
                mrow = msgs_v[e, pl.ds(k * 16, 16)]
                arow = agg_v[pl.ds(base + k * 16, 16)]
                agg_v[pl.ds(base + k * 16, 16)] = jnp.maximum(arow, mrow)

        def _group(j, _):
            gb = j * 16
            dv = accd_v[pl.ds(gb, 16)]
            for t in range(16):
                _rmw1(gb + t, dv[t])
            return 0
        lax.fori_loop(0, n // 16, _group, 0)

        def _edge(e, _):
            d = accd_v[pl.ds(e, 16)][0]
            _rmw1(e, d)
            return 0
        lax.fori_loop((n // 16) * 16, n, _edge, 0)

    def _issue(cc, dbuf, sbuf, sm):
        pltpu.async_copy(dst_h.at[pl.ds(cc * CHK, CHK)], dbuf, sm)
        pltpu.async_copy(src_h.at[pl.ds(cc * CHK, CHK)], sbuf, sm)

    def _wait(cc, dbuf, sbuf, sm):
        pltpu.make_async_copy(dst_h.at[pl.ds(cc * CHK, CHK)], dbuf, sm).wait()
        pltpu.make_async_copy(src_h.at[pl.ds(cc * CHK, CHK)], sbuf, sm).wait()

    _pidx = [jnp.maximum(lanes - sh, 0) for sh in (1, 2, 4, 8)]
    _pmsk = [lanes >= sh for sh in (1, 2, 4, 8)]
    _l15 = jnp.full((16,), 15, jnp.int32)

    def _scan_chunk(dbuf, sbuf, offv0):
        def _scan16(i5, offv):
            for u in range(25):
                i = i5 * 25 + u
                dvec = dbuf[pl.ds(i * 16, 16)]
                m = (dvec >= lo) & (dvec < hi)
                p = m.astype(jnp.int32)
                for ci, cm in zip(_pidx, _pmsk):
                    p = p + jnp.where(cm, p[ci], 0)
                idxv = p - 1 + offv
                svec = sbuf[pl.ds(i * 16, 16)]
                wvec = plsc.load_gather(wid_v, [svec], mask=m)
                plsc.store_scatter(accd_v, [idxv], dvec - lo, mask=m)
                plsc.store_scatter(accw_v, [idxv], wvec, mask=m)
                offv = offv + p[_l15]
            return offv
        return lax.fori_loop(0, CHK // 400, _scan16, offv0)

    def _drain(j, off):
        _process_batch(BB)

        def _mv(i, _):
            accd_v[pl.ds(i * 16, 16)] = accd_v[pl.ds(BB + i * 16, 16)]
            accw_v[pl.ds(i * 16, 16)] = accw_v[pl.ds(BB + i * 16, 16)]
            return 0
        lax.fori_loop(0, (ACCN - BB) // 16, _mv, 0)
        return off - BB

    NCH = E // CHK
    _issue(0, dstc_v, srcc_v, sem2)

    def _chunk2(c2, offv):
        for ph in range(2):
            cc = c2 * 2 + ph
            if ph == 0:
                db, sb, sm = dstc_v, srcc_v, sem2
                ndb, nsb, nsm = dstc2_v, srcc2_v, sem3
            else:
                db, sb, sm = dstc2_v, srcc2_v, sem3
                ndb, nsb, nsm = dstc_v, srcc_v, sem2
            _wait(cc, db, sb, sm)

            @pl.when(cc + 1 < NCH)
            def _(cc=cc, ndb=ndb, nsb=nsb, nsm=nsm):
                _issue(cc + 1, ndb, nsb, nsm)
            offv = _scan_chunk(db, sb, offv)
            off_sc = offv[0]
            ndr = off_sc // BB
            lax.fori_loop(0, ndr, _drain, off_sc)
            offv = offv - ndr * BB
        return offv

    offv = lax.fori_loop(0, NCH // 2, _chunk2, jnp.zeros((16,), jnp.int32))
    _process_batch(offv[0])

    def _cpw(i, _):
        widb_v[pl.ds(i * 16, 16)] = wid_v[pl.ds(lo + i * 16, 16)]
        return 0
    lax.fori_loop(0, (R + 15) // 16, _cpw, 0)
    pltpu.async_copy(tab_h.at[widb_v], msgs_v, sem).wait()

    def _rst(r, _):
        base = r * D
        for k in range(8):
            a = agg_v[pl.ds(base + k * 16, 16)]
            h = msgs_v[r, pl.ds(k * 16, 16)]
            fixed = jnp.where(a == -jnp.inf, 0.0, a)
            agg_v[pl.ds(base + k * 16, 16)] = fixed + h
        return 0
    lax.fori_loop(0, R, _rst, 0)

    pltpu.sync_copy(agg_v, rst_h.at[pl.ds(lo * D, R * D)])


def _mm_body(x_ref, wt_ref, b_ref, o_ref):
    o_ref[...] = (
        jnp.dot(x_ref[...], wt_ref[...], preferred_element_type=jnp.float32)
        + b_ref[...]
    )


def _readout_body(res_h, gid_h, out_h, gid_v, outl_v, resc_v, sem):
    w = lax.axis_index("s") * 2 + lax.axis_index("c")
    glo = w * 8
    neg16 = jnp.full((16,), -jnp.inf, jnp.float32)

    pltpu.sync_copy(gid_h, gid_v.at[pl.ds(0, N)])

    def _cnt(i, carry):
        a, b = carry
        g = gid_v[pl.ds(i * 16, 16)]
        a = a + plsc.all_reduce_population_count(g < glo)[0]
        b = b + plsc.all_reduce_population_count(g < glo + 8)[0]
        return (a, b)
    start, end = lax.fori_loop(0, N // 16, _cnt, (0, 0))

    for t in range(64):
        outl_v[pl.ds(t * 16, 16)] = neg16

    astart = (start // 8) * 8
    nch = (end - astart + KC - 1) // KC

    def _ch(jc, _):
        c0 = astart + jc * KC
        pltpu.sync_copy(res_h.at[pl.ds(c0, KC)], resc_v)
        j0 = jnp.maximum(start - c0, 0)
        j1 = jnp.minimum(end - c0, KC)

        def _node(j, _):
            gg = gid_v[pl.ds(c0 + j, 16)][0]
            base = (gg - glo) * D
            for k in range(8):
                r = resc_v[j, pl.ds(k * 16, 16)]
                o = outl_v[pl.ds(base + k * 16, 16)]
                outl_v[pl.ds(base + k * 16, 16)] = jnp.maximum(o, r)
            return 0
        lax.fori_loop(j0, j1, _node, 0)
        return 0
    lax.fori_loop(0, nch, _ch, 0)

    for t in range(64):
        v = outl_v[pl.ds(t * 16, 16)]
        outl_v[pl.ds(t * 16, 16)] = jnp.where(v == -jnp.inf, 0.0, v)
    pltpu.sync_copy(outl_v, out_h.at[pl.ds(glo * D, 8 * D)])


_SC_MESH = plsc.VectorSubcoreMesh(core_axis_name="c", subcore_axis_name="s")

_agg_call = functools.partial(
    pl.kernel,
    _agg_body,
    out_type=jax.ShapeDtypeStruct((NPAD * D,), jnp.float32),
    mesh=_SC_MESH,
    compiler_params=pltpu.CompilerParams(needs_layout_passes=False),
    scratch_types=[
        pltpu.VMEM((N + 32,), jnp.int32),
        pltpu.VMEM((R * D,), jnp.float32),
        pltpu.VMEM((BB, D), jnp.float32),
        pltpu.VMEM((CHK,), jnp.int32),
        pltpu.VMEM((CHK,), jnp.int32),
        pltpu.VMEM((CHK,), jnp.int32),
        pltpu.VMEM((CHK,), jnp.int32),
        pltpu.VMEM((ACCN,), jnp.int32),
        pltpu.VMEM((ACCN,), jnp.int32),
        pltpu.VMEM((BB,), jnp.int32),
        pltpu.VMEM((R + 15,), jnp.int32),
        pltpu.SemaphoreType.DMA,
        pltpu.SemaphoreType.DMA,
        pltpu.SemaphoreType.DMA,
    ],
)()

_readout_call = functools.partial(
    pl.kernel,
    _readout_body,
    out_type=jax.ShapeDtypeStruct((G * D,), jnp.float32),
    mesh=_SC_MESH,
    compiler_params=pltpu.CompilerParams(needs_layout_passes=False),
    scratch_types=[
        pltpu.VMEM((NPAD,), jnp.int32),
        pltpu.VMEM((8 * D,), jnp.float32),
        pltpu.VMEM((KC, D), jnp.float32),
        pltpu.SemaphoreType.DMA,
    ],
)()

_mm_call = pl.pallas_call(
    _mm_body,
    grid=(5,),
    in_specs=[
        pl.BlockSpec((2000, D), lambda i: (i, 0)),
        pl.BlockSpec((D, D), lambda i: (0, 0)),
        pl.BlockSpec((1, D), lambda i: (0, 0)),
    ],
    out_specs=pl.BlockSpec((2000, D), lambda i: (i, 0)),
    out_shape=jax.ShapeDtypeStruct((N, D), jnp.float32),
)


def kernel(wid, edge_index, graph_ids, emb_table, W, b):
    src = edge_index[0]
    dst = edge_index[1]
    rst_flat = _agg_call(wid, src, dst, emb_table)
    rst = rst_flat.reshape(NPAD, D)[:N]
    res = _mm_call(rst, W.T, b.reshape(1, D))
    res_pad = jnp.pad(res, ((0, NPAD_C - N), (0, 0)))
    out_flat = _readout_call(res_pad, graph_ids)
    return out_flat.reshape(G, D)


# speedup vs baseline: 1.0029x; 1.0029x over previous
import functools

import jax
import jax.numpy as jnp
from jax import lax
from jax.experimental import pallas as pl
from jax.experimental.pallas import tpu as pltpu
from jax.experimental.pallas import tpu_sc as plsc

N = 10000
E = 320000
D = 128
V = 1000
G = 256

NW = 32
R = 313
NPAD = NW * R
CHK = 2000
BB = 512
ACCN = BB + CHK + 16

KC = 256
NPAD_C = N + KC


def _agg_body(wid_h, src_h, dst_h, tab_h, rst_h,
              wid_v, agg_v, msgs_v, dstc_v, srcc_v, dstc2_v, srcc2_v,
              accd_v, accw_v, widb_v, tmp_v, sem, sem2, sem3):
    w = lax.axis_index("s") * 2 + lax.axis_index("c")
    lo = w * R
    hi = jnp.minimum(lo + R, N)
    neg16 = jnp.full((16,), -jnp.inf, jnp.float32)
    zero16 = jnp.zeros((16,), jnp.int32)

    pltpu.sync_copy(wid_h, wid_v.at[pl.ds(0, N)])
    wid_v[pl.ds(N, 16)] = zero16
    wid_v[pl.ds(N + 16, 16)] = zero16

    def _init_accw(i, _):
        accw_v[pl.ds(i * 16, 16)] = zero16
        return 0
    lax.fori_loop(0, ACCN // 16, _init_accw, 0)

    def _init_agg(i, _):
        agg_v[pl.ds(i * 16, 16)] = neg16
        return 0
    lax.fori_loop(0, (R * D) // 16, _init_agg, 0)

    lanes = lax.iota(jnp.int32, 16)

    def _process_batch(n):
        def _cp(i, _):
            widb_v[pl.ds(i * 16, 16)] = accw_v[pl.ds(i * 16, 16)]
            return 0
        lax.fori_loop(0, BB // 16, _cp, 0)
        pltpu.async_copy(tab_h.at[widb_v], msgs_v, sem).wait()

        def _rmw1(e, d):
            base = d * D
            for k in range(8):
                mrow = msgs_v[e, pl.ds(k * 16, 16)]
                arow = agg_v[pl.ds(base + k * 16, 16)]
                agg_v[pl.ds(base + k * 16, 16)] = jnp.maximum(arow, mrow)

        def _group(j, _):
            gb = j * 16
            dv = accd_v[pl.ds(gb, 16)]
            for t in range(16):
                _rmw1(gb + t, dv[t])
            return 0
        lax.fori_loop(0, n // 16, _group, 0)

        def _edge(e, _):
            d = accd_v[pl.ds(e, 16)][0]
            _rmw1(e, d)
            return 0
        lax.fori_loop((n // 16) * 16, n, _edge, 0)

    def _issue(cc, dbuf, sbuf, sm):
        pltpu.async_copy(dst_h.at[pl.ds(cc * CHK, CHK)], dbuf, sm)
        pltpu.async_copy(src_h.at[pl.ds(cc * CHK, CHK)], sbuf, sm)

    def _wait(cc, dbuf, sbuf, sm):
        pltpu.make_async_copy(dst_h.at[pl.ds(cc * CHK, CHK)], dbuf, sm).wait()
        pltpu.make_async_copy(src_h.at[pl.ds(cc * CHK, CHK)], sbuf, sm).wait()

    _pidx = [jnp.maximum(lanes - sh, 0) for sh in (1, 2, 4, 8)]
    _pmsk = [lanes >= sh for sh in (1, 2, 4, 8)]
    _l15 = jnp.full((16,), 15, jnp.int32)

    def _scan_chunk(dbuf, sbuf, offv0):
        def _scan16(i5, offv):
            for u in range(5):
                i = i5 * 5 + u
                dvec = dbuf[pl.ds(i * 16, 16)]
                m = (dvec >= lo) & (dvec < hi)
                p = m.astype(jnp.int32)
                for ci, cm in zip(_pidx, _pmsk):
                    p = p + jnp.where(cm, p[ci], 0)
                idxv = p - 1 + offv
                svec = sbuf[pl.ds(i * 16, 16)]
                wvec = plsc.load_gather(wid_v, [svec], mask=m)
                plsc.store_scatter(accd_v, [idxv], dvec - lo, mask=m)
                plsc.store_scatter(accw_v, [idxv], wvec, mask=m)
                offv = offv + p[_l15]
            return offv
        return lax.fori_loop(0, CHK // 80, _scan16, offv0)

    def _drain(j, off):
        _process_batch(BB)

        def _mv(i, _):
            accd_v[pl.ds(i * 16, 16)] = accd_v[pl.ds(BB + i * 16, 16)]
            accw_v[pl.ds(i * 16, 16)] = accw_v[pl.ds(BB + i * 16, 16)]
            return 0
        lax.fori_loop(0, (ACCN - BB) // 16, _mv, 0)
        return off - BB

    NCH = E // CHK
    _issue(0, dstc_v, srcc_v, sem2)

    def _chunk2(c2, offv):
        for ph in range(2):
            cc = c2 * 2 + ph
            if ph == 0:
                db, sb, sm = dstc_v, srcc_v, sem2
                ndb, nsb, nsm = dstc2_v, srcc2_v, sem3
            else:
                db, sb, sm = dstc2_v, srcc2_v, sem3
                ndb, nsb, nsm = dstc_v, srcc_v, sem2
            _wait(cc, db, sb, sm)

            @pl.when(cc + 1 < NCH)
            def _(cc=cc, ndb=ndb, nsb=nsb, nsm=nsm):
                _issue(cc + 1, ndb, nsb, nsm)
            offv = _scan_chunk(db, sb, offv)
            off_sc = offv[0]
            ndr = off_sc // BB
            lax.fori_loop(0, ndr, _drain, off_sc)
            offv = offv - ndr * BB
        return offv

    offv = lax.fori_loop(0, NCH // 2, _chunk2, jnp.zeros((16,), jnp.int32))
    _process_batch(offv[0])

    def _cpw(i, _):
        widb_v[pl.ds(i * 16, 16)] = wid_v[pl.ds(lo + i * 16, 16)]
        return 0
    lax.fori_loop(0, (R + 15) // 16, _cpw, 0)
    pltpu.async_copy(tab_h.at[widb_v], msgs_v, sem).wait()

    def _rst(r, _):
        base = r * D
        for k in range(8):
            a = agg_v[pl.ds(base + k * 16, 16)]
            h = msgs_v[r, pl.ds(k * 16, 16)]
            fixed = jnp.where(a == -jnp.inf, 0.0, a)
            agg_v[pl.ds(base + k * 16, 16)] = fixed + h
        return 0
    lax.fori_loop(0, R, _rst, 0)

    pltpu.sync_copy(agg_v, rst_h.at[pl.ds(lo * D, R * D)])


def _mm_body(x_ref, wt_ref, b_ref, o_ref):
    o_ref[...] = (
        jnp.dot(x_ref[...], wt_ref[...], preferred_element_type=jnp.float32)
        + b_ref[...]
    )


def _readout_body(res_h, gid_h, out_h, gid_v, outl_v, resc_v, sem):
    w = lax.axis_index("s") * 2 + lax.axis_index("c")
    glo = w * 8
    neg16 = jnp.full((16,), -jnp.inf, jnp.float32)

    pltpu.sync_copy(gid_h, gid_v.at[pl.ds(0, N)])

    def _cnt(i, carry):
        a, b = carry
        g = gid_v[pl.ds(i * 16, 16)]
        a = a + plsc.all_reduce_population_count(g < glo)[0]
        b = b + plsc.all_reduce_population_count(g < glo + 8)[0]
        return (a, b)
    start, end = lax.fori_loop(0, N // 16, _cnt, (0, 0))

    for t in range(64):
        outl_v[pl.ds(t * 16, 16)] = neg16

    astart = (start // 8) * 8
    nch = (end - astart + KC - 1) // KC

    def _ch(jc, _):
        c0 = astart + jc * KC
        pltpu.sync_copy(res_h.at[pl.ds(c0, KC)], resc_v)
        j0 = jnp.maximum(start - c0, 0)
        j1 = jnp.minimum(end - c0, KC)

        def _node(j, _):
            gg = gid_v[pl.ds(c0 + j, 16)][0]
            base = (gg - glo) * D
            for k in range(8):
                r = resc_v[j, pl.ds(k * 16, 16)]
                o = outl_v[pl.ds(base + k * 16, 16)]
                outl_v[pl.ds(base + k * 16, 16)] = jnp.maximum(o, r)
            return 0
        lax.fori_loop(j0, j1, _node, 0)
        return 0
    lax.fori_loop(0, nch, _ch, 0)

    for t in range(64):
        v = outl_v[pl.ds(t * 16, 16)]
        outl_v[pl.ds(t * 16, 16)] = jnp.where(v == -jnp.inf, 0.0, v)
    pltpu.sync_copy(outl_v, out_h.at[pl.ds(glo * D, 8 * D)])


_SC_MESH = plsc.VectorSubcoreMesh(core_axis_name="c", subcore_axis_name="s")

_agg_call = functools.partial(
    pl.kernel,
    _agg_body,
    out_type=jax.ShapeDtypeStruct((NPAD * D,), jnp.float32),
    mesh=_SC_MESH,
    compiler_params=pltpu.CompilerParams(needs_layout_passes=False),
    scratch_types=[
        pltpu.VMEM((N + 32,), jnp.int32),
        pltpu.VMEM((R * D,), jnp.float32),
        pltpu.VMEM((BB, D), jnp.float32),
        pltpu.VMEM((CHK,), jnp.int32),
        pltpu.VMEM((CHK,), jnp.int32),
        pltpu.VMEM((CHK,), jnp.int32),
        pltpu.VMEM((CHK,), jnp.int32),
        pltpu.VMEM((ACCN,), jnp.int32),
        pltpu.VMEM((ACCN,), jnp.int32),
        pltpu.VMEM((BB,), jnp.int32),
        pltpu.VMEM((R + 15,), jnp.int32),
        pltpu.SemaphoreType.DMA,
        pltpu.SemaphoreType.DMA,
        pltpu.SemaphoreType.DMA,
    ],
)()

_readout_call = functools.partial(
    pl.kernel,
    _readout_body,
    out_type=jax.ShapeDtypeStruct((G * D,), jnp.float32),
    mesh=_SC_MESH,
    compiler_params=pltpu.CompilerParams(needs_layout_passes=False),
    scratch_types=[
        pltpu.VMEM((NPAD,), jnp.int32),
        pltpu.VMEM((8 * D,), jnp.float32),
        pltpu.VMEM((KC, D), jnp.float32),
        pltpu.SemaphoreType.DMA,
    ],
)()

_mm_call = pl.pallas_call(
    _mm_body,
    grid=(5,),
    in_specs=[
        pl.BlockSpec((2000, D), lambda i: (i, 0)),
        pl.BlockSpec((D, D), lambda i: (0, 0)),
        pl.BlockSpec((1, D), lambda i: (0, 0)),
    ],
    out_specs=pl.BlockSpec((2000, D), lambda i: (i, 0)),
    out_shape=jax.ShapeDtypeStruct((N, D), jnp.float32),
)


def kernel(wid, edge_index, graph_ids, emb_table, W, b):
    src = edge_index[0]
    dst = edge_index[1]
    rst_flat = _agg_call(wid, src, dst, emb_table)
    rst = rst_flat.reshape(NPAD, D)[:N]
    res = _mm_call(rst, W.T, b.reshape(1, D))
    res_pad = jnp.pad(res, ((0, NPAD_C - N), (0, 0)))
    out_flat = _readout_call(res_pad, graph_ids)
    return out_flat.reshape(G, D)
